# baseline (speedup 1.0000x reference)
import functools

import jax
import jax.numpy as jnp
from jax import lax
from jax.experimental import pallas as pl
from jax.experimental.pallas import tpu as pltpu
from jax.experimental.pallas import tpu_sc as plsc

_N = 50000
_E = 800000
_DIN = 128
_H = 64
_HH = 32
_OUT = 8
_LAYERS = 5
_ORDER = 2
_ITERS = 8

_NP = 50176
_RPS = 3136
_ZR = 56
_CH = 125
_KG = 2
_NCHUNK = _E // _CH
_CPS = _NCHUNK // 16
_GPS = _CPS // _KG
_GI = 5
_GO = _GPS // _GI

_BLK = 2000
_GRID = _N // _BLK


def _leaky(x):
    return jnp.where(x >= 0, x, 0.2 * x)


def _sc_prop(hs0, hs1, src2d, dst2d):
    mesh = plsc.VectorSubcoreMesh(core_axis_name="c", subcore_axis_name="s")

    @functools.partial(
        pl.kernel,
        mesh=mesh,
        compiler_params=pltpu.CompilerParams(use_tc_tiling_on_sc=False),
        out_type=[
            jax.ShapeDtypeStruct((_NP, _HH), jnp.float32),
            jax.ShapeDtypeStruct((_NP, _HH), jnp.float32),
        ],
        scratch_types=[
            pltpu.VMEM_SHARED((_NP, _HH), jnp.float32),
            pltpu.VMEM((_GI * _KG, _CH), jnp.int32),
            pltpu.VMEM((_GI * _KG, _CH), jnp.int32),
            pltpu.VMEM((3 * _KG, _CH, _HH), jnp.float32),
            pltpu.VMEM((_ZR, _HH), jnp.float32),
            pltpu.SemaphoreType.DMA,
            pltpu.SemaphoreType.DMA,
            pltpu.SemaphoreType.DMA,
            pltpu.SemaphoreType.DMA,
            pltpu.SemaphoreType.DMA,
            pltpu.SemaphoreType.DMA,
            pltpu.SemaphoreType.DMA,
        ],
    )
    def kern(hs0_hbm, hs1_hbm, src_hbm, dst_hbm, o0_hbm, o1_hbm,
             acc, sidx, didx, rows, zbuf, sem_i,
             sem_g0, sem_g1, sem_g2, sem_s0, sem_s1, sem_s2):
        cid = lax.axis_index("c")
        sid = lax.axis_index("s")
        sem_g = (sem_g0, sem_g1, sem_g2)
        sem_s = (sem_s0, sem_s1, sem_s2)

        @pl.loop(0, _ZR)
        def _(r):
            zbuf[r, pl.ds(0, 16)] = jnp.zeros((16,), jnp.float32)
            zbuf[r, pl.ds(16, 16)] = jnp.zeros((16,), jnp.float32)

        def run(hs_hbm, o_hbm):
            base = sid * _RPS
            zcps = [
                pltpu.async_copy(zbuf, acc.at[pl.ds(base + i * _ZR, _ZR)],
                                 sem_s0)
                for i in range(_RPS // _ZR)
            ]
            for c in zcps:
                c.wait()

            plsc.subcore_barrier()

            c00 = sid * _CPS

            def fire_gath(g):
                p = g % 3
                return [
                    pltpu.async_copy(
                        hs_hbm.at[sidx.at[g * _KG + j]],
                        rows.at[p * _KG + j], sem_g[p])
                    for j in range(_KG)
                ]

            def fire_scat(g):
                p = g % 3
                return [
                    pltpu.async_copy(
                        rows.at[p * _KG + j],
                        acc.at[didx.at[g * _KG + j]],
                        sem_s[p], add=True)
                    for j in range(_KG)
                ]

            @pl.loop(0, _GO)
            def _(gi):
                gbase = c00 + gi * (_GI * _KG)
                ci_s = pltpu.async_copy(
                    src_hbm.at[pl.ds(gbase, _GI * _KG)], sidx, sem_i)
                ci_d = pltpu.async_copy(
                    dst_hbm.at[pl.ds(gbase, _GI * _KG)], didx, sem_i)
                ci_s.wait()
                ci_d.wait()
                gath = {0: fire_gath(0)}
                scat = {}
                for g in range(_GI):
                    if g + 1 < _GI:
                        if g - 2 >= 0:
                            for c in scat[g - 2]:
                                c.wait()
                        gath[g + 1] = fire_gath(g + 1)
                    for c in gath[g]:
                        c.wait()
                    scat[g] = fire_scat(g)
                for g in range(max(0, _GI - 3), _GI):
                    for c in scat[g]:
                        c.wait()

            plsc.subcore_barrier()
            pltpu.sync_copy(acc.at[pl.ds(sid * _RPS, _RPS)],
                            o_hbm.at[pl.ds(sid * _RPS, _RPS)])

        @pl.when(cid == 0)
        def _():
            run(hs0_hbm, o0_hbm)

        @pl.when(cid == 1)
        def _():
            run(hs1_hbm, o1_hbm)

    return kern(hs0, hs1, src2d, dst2d)


def _row_spec(cols):
    return pl.BlockSpec((_BLK, cols), lambda i: (i, 0))


def _full_spec(shape):
    return pl.BlockSpec(shape, lambda i: tuple(0 for _ in shape))


def _t1_body(x, w1, b1, w2, b2, invs, xo, hs0, hs1):
    h = _leaky(jnp.dot(x[...], w1[...], preferred_element_type=jnp.float32)
               + b1[...])
    h = _leaky(jnp.dot(h, w2[...], preferred_element_type=jnp.float32)
               + b2[...])
    xo[...] = h
    hs = invs[...] * h
    hs0[...] = hs[:, :_HH]
    hs1[...] = hs[:, _HH:]


def _t1(x, w1, b1, w2, b2, invs):
    f32 = jnp.float32
    return pl.pallas_call(
        _t1_body,
        grid=(_GRID,),
        in_specs=[_row_spec(_DIN), _full_spec((_DIN, _H)), _full_spec((1, _H)),
                  _full_spec((_H, _H)), _full_spec((1, _H)), _row_spec(1)],
        out_specs=[_row_spec(_H), _row_spec(_HH), _row_spec(_HH)],
        out_shape=[jax.ShapeDtypeStruct((_N, _H), f32),
                   jax.ShapeDtypeStruct((_N, _HH), f32),
                   jax.ShapeDtypeStruct((_N, _HH), f32)],
    )(x, w1, b1, w2, b2, invs)


def _t2_body(x, v00, b00, v10, b10, v01, b01, v11, b11,
             o00, o10, o01, o11):
    xv = x[...]
    o00[...] = jnp.dot(xv, v00[...], preferred_element_type=jnp.float32) + b00[...]
    o10[...] = jnp.dot(xv, v10[...], preferred_element_type=jnp.float32) + b10[...]
    o01[...] = jnp.dot(xv, v01[...], preferred_element_type=jnp.float32) + b01[...]
    o11[...] = jnp.dot(xv, v11[...], preferred_element_type=jnp.float32) + b11[...]


def _t2(x, v00, b00, v10, b10, v01, b01, v11, b11):
    f32 = jnp.float32
    return pl.pallas_call(
        _t2_body,
        grid=(_GRID,),
        in_specs=[_row_spec(_H)] + [_full_spec((_H, _H)), _full_spec((1, _H))] * 4,
        out_specs=[_row_spec(_H)] * 4,
        out_shape=[jax.ShapeDtypeStruct((_N, _H), f32)] * 4,
    )(x, v00, b00, v10, b10, v01, b01, v11, b11)


def _t3_body(a0, a1, w, xv, invd, invs, h_o, hs0_o, hs1_o):
    wm = w[...]
    t = (jnp.dot(a0[...], wm[:_HH, :], preferred_element_type=jnp.float32)
         + jnp.dot(a1[...], wm[_HH:, :], preferred_element_type=jnp.float32))
    h = _leaky(invd[...] * t + xv[...])
    h_o[...] = h
    hs = invs[...] * h
    hs0_o[...] = hs[:, :_HH]
    hs1_o[...] = hs[:, _HH:]


def _t3(a0, a1, w, xv, invd, invs):
    f32 = jnp.float32
    return pl.pallas_call(
        _t3_body,
        grid=(_GRID,),
        in_specs=[_row_spec(_HH), _row_spec(_HH), _full_spec((_H, _H)),
                  _row_spec(_H), _row_spec(1), _row_spec(1)],
        out_specs=[_row_spec(_H), _row_spec(_HH), _row_spec(_HH)],
        out_shape=[jax.ShapeDtypeStruct((_N, _H), f32),
                   jax.ShapeDtypeStruct((_N, _HH), f32),
                   jax.ShapeDtypeStruct((_N, _HH), f32)],
    )(a0, a1, w, xv, invd, invs)


def _t4_body(h0, h1, invs, xo, hs0, hs1):
    x = (h0[...] + h1[...]) * 0.5
    xo[...] = x
    hs = invs[...] * x
    hs0[...] = hs[:, :_HH]
    hs1[...] = hs[:, _HH:]


def _t4(h0, h1, invs):
    f32 = jnp.float32
    return pl.pallas_call(
        _t4_body,
        grid=(_GRID,),
        in_specs=[_row_spec(_H), _row_spec(_H), _row_spec(1)],
        out_specs=[_row_spec(_H), _row_spec(_HH), _row_spec(_HH)],
        out_shape=[jax.ShapeDtypeStruct((_N, _H), f32),
                   jax.ShapeDtypeStruct((_N, _HH), f32),
                   jax.ShapeDtypeStruct((_N, _HH), f32)],
    )(h0, h1, invs)


def _t5_body(h0, h1, w1, b1, w2, b2, wr, br, out):
    x = (h0[...] + h1[...]) * 0.5
    x = _leaky(jnp.dot(x, w1[...], preferred_element_type=jnp.float32) + b1[...])
    x = _leaky(jnp.dot(x, w2[...], preferred_element_type=jnp.float32) + b2[...])
    out[...] = jnp.dot(x, wr[...], preferred_element_type=jnp.float32) + br[...]


def _t5(h0, h1, w1, b1, w2, b2, wr, br):
    return pl.pallas_call(
        _t5_body,
        grid=(_GRID,),
        in_specs=[_row_spec(_H), _row_spec(_H),
                  _full_spec((_H, _H)), _full_spec((1, _H)),
                  _full_spec((_H, _H)), _full_spec((1, _H)),
                  _full_spec((_H, _OUT)), _full_spec((1, _OUT))],
        out_specs=[_row_spec(_OUT)],
        out_shape=[jax.ShapeDtypeStruct((_N, _OUT), jnp.float32)],
    )(h0, h1, w1, b1, w2, b2, wr, br)[0]


def kernel(X, edge_index, pre1_W, pre1_b, pre2_W, pre2_b,
           arma_W0, arma_V0, arma_b0, arma_W1, arma_V1, arma_b1,
           post1_W, post1_b, post2_W, post2_b, ro_W, ro_b):
    f32 = jnp.float32
    src = edge_index[0]
    dst = edge_index[1]
    deg_s = jnp.zeros((_N,), f32).at[src].add(1.0)
    deg_d = jnp.zeros((_N,), f32).at[dst].add(1.0)
    inv_s = jnp.where(deg_s > 0, lax.rsqrt(jnp.maximum(deg_s, 1.0)), 0.0)
    inv_d = jnp.where(deg_d > 0, lax.rsqrt(jnp.maximum(deg_d, 1.0)), 0.0)
    invs2 = inv_s[:, None]
    invd2 = inv_d[:, None]

    src_s, dst_s = lax.sort([src, dst], num_keys=1)
    src2d = src_s.reshape(_NCHUNK, _CH)
    dst2d = dst_s.reshape(_NCHUNK, _CH)

    def b2d(b):
        return b.reshape(1, -1)

    x_in, hs0, hs1 = _t1(X, pre1_W, b2d(pre1_b), pre2_W, b2d(pre2_b), invs2)

    for l in range(_LAYERS):
        xv00, xv10, xv01, xv11 = _t2(
            x_in,
            arma_V0[l, 0], b2d(arma_b0[l, 0]), arma_V1[l, 0], b2d(arma_b1[l, 0]),
            arma_V0[l, 1], b2d(arma_b0[l, 1]), arma_V1[l, 1], b2d(arma_b1[l, 1]))
        xv0 = (xv00, xv10)
        xv1 = (xv01, xv11)
        hs = [(hs0, hs1), (hs0, hs1)]
        h = [None, None]
        for t in range(_ITERS):
            for k in range(_ORDER):
                a0, a1 = _sc_prop(hs[k][0], hs[k][1], src2d, dst2d)
                W = (arma_W0 if t == 0 else arma_W1)[l, k]
                xv = (xv0 if k == 0 else xv1)[0 if t == 0 else 1]
                h[k], h0_, h1_ = _t3(a0, a1, W, xv, invd2, invs2)
                hs[k] = (h0_, h1_)
        if l < _LAYERS - 1:
            x_in, hs0, hs1 = _t4(h[0], h[1], invs2)

    return _t5(h[0], h[1], post1_W, b2d(post1_b), post2_W, b2d(post2_b),
               ro_W, b2d(ro_b))


# speedup vs baseline: 1.5676x; 1.5676x over previous
import functools

import jax
import jax.numpy as jnp
from jax import lax
from jax.experimental import pallas as pl
from jax.experimental.pallas import tpu as pltpu
from jax.experimental.pallas import tpu_sc as plsc

_N = 50000
_E = 800000
_DIN = 128
_H = 64
_HH = 32
_OUT = 8
_LAYERS = 5
_ORDER = 2
_ITERS = 8

_NP = 50176
_RPS = 3136
_ZR = 56
_CH = 100
_KG = 2
_NPAR = 4
_NCHUNK = _E // _CH
_CPS = _NCHUNK // 16
_GPS = _CPS // _KG
_GI = 5
_GO = _GPS // _GI

_BLK = 2000
_GRID = _N // _BLK


def _leaky(x):
    return jnp.where(x >= 0, x, 0.2 * x)


def _sc_prop(hs0, hs1, src2d, dst2d):
    mesh = plsc.VectorSubcoreMesh(core_axis_name="c", subcore_axis_name="s")

    @functools.partial(
        pl.kernel,
        mesh=mesh,
        compiler_params=pltpu.CompilerParams(use_tc_tiling_on_sc=False),
        out_type=[
            jax.ShapeDtypeStruct((_NP, _HH), jnp.float32),
            jax.ShapeDtypeStruct((_NP, _HH), jnp.float32),
        ],
        scratch_types=[
            pltpu.VMEM_SHARED((_NP, _HH), jnp.float32),
            pltpu.VMEM((_GI * _KG, _CH), jnp.int32),
            pltpu.VMEM((_GI * _KG, _CH), jnp.int32),
            pltpu.VMEM((_NPAR * _KG, _CH, _HH), jnp.float32),
            pltpu.VMEM((_ZR, _HH), jnp.float32),
            pltpu.SemaphoreType.DMA,
            pltpu.SemaphoreType.DMA,
            pltpu.SemaphoreType.DMA,
            pltpu.SemaphoreType.DMA,
            pltpu.SemaphoreType.DMA,
            pltpu.SemaphoreType.DMA,
            pltpu.SemaphoreType.DMA,
            pltpu.SemaphoreType.DMA,
            pltpu.SemaphoreType.DMA,
        ],
    )
    def kern(hs0_hbm, hs1_hbm, src_hbm, dst_hbm, o0_hbm, o1_hbm,
             acc, sidx, didx, rows, zbuf, sem_i,
             sem_g0, sem_g1, sem_g2, sem_g3,
             sem_s0, sem_s1, sem_s2, sem_s3):
        cid = lax.axis_index("c")
        sid = lax.axis_index("s")
        sem_g = (sem_g0, sem_g1, sem_g2, sem_g3)
        sem_s = (sem_s0, sem_s1, sem_s2, sem_s3)

        @pl.loop(0, _ZR)
        def _(r):
            zbuf[r, pl.ds(0, 16)] = jnp.zeros((16,), jnp.float32)
            zbuf[r, pl.ds(16, 16)] = jnp.zeros((16,), jnp.float32)

        def run(hs_hbm, o_hbm):
            base = sid * _RPS
            zcps = [
                pltpu.async_copy(zbuf, acc.at[pl.ds(base + i * _ZR, _ZR)],
                                 sem_s0)
                for i in range(_RPS // _ZR)
            ]
            for c in zcps:
                c.wait()

            plsc.subcore_barrier()

            c00 = sid * _CPS

            def fire_gath(g):
                p = g % _NPAR
                return [
                    pltpu.async_copy(
                        hs_hbm.at[sidx.at[g * _KG + j]],
                        rows.at[p * _KG + j], sem_g[p])
                    for j in range(_KG)
                ]

            def fire_scat(g):
                p = g % _NPAR
                return [
                    pltpu.async_copy(
                        rows.at[p * _KG + j],
                        acc.at[didx.at[g * _KG + j]],
                        sem_s[p], add=True)
                    for j in range(_KG)
                ]

            @pl.loop(0, _GO)
            def _(gi):
                gbase = c00 + gi * (_GI * _KG)
                ci_s = pltpu.async_copy(
                    src_hbm.at[pl.ds(gbase, _GI * _KG)], sidx, sem_i)
                ci_d = pltpu.async_copy(
                    dst_hbm.at[pl.ds(gbase, _GI * _KG)], didx, sem_i)
                ci_s.wait()
                ci_d.wait()
                gath = {0: fire_gath(0), 1: fire_gath(1)}
                scat = {}
                waited = -1
                for g in range(_GI):
                    if g + 2 < _GI:
                        if g - 2 >= 0:
                            for c in scat[g - 2]:
                                c.wait()
                            waited = g - 2
                        gath[g + 2] = fire_gath(g + 2)
                    for c in gath[g]:
                        c.wait()
                    scat[g] = fire_scat(g)
                for g in range(waited + 1, _GI):
                    for c in scat[g]:
                        c.wait()

            plsc.subcore_barrier()
            pltpu.sync_copy(acc.at[pl.ds(sid * _RPS, _RPS)],
                            o_hbm.at[pl.ds(sid * _RPS, _RPS)])

        @pl.when(cid == 0)
        def _():
            run(hs0_hbm, o0_hbm)

        @pl.when(cid == 1)
        def _():
            run(hs1_hbm, o1_hbm)

    return kern(hs0, hs1, src2d, dst2d)


def _row_spec(cols):
    return pl.BlockSpec((_BLK, cols), lambda i: (i, 0))


def _full_spec(shape):
    return pl.BlockSpec(shape, lambda i: tuple(0 for _ in shape))


def _t1_body(x, w1, b1, w2, b2, invs, xo, hs0, hs1):
    h = _leaky(jnp.dot(x[...], w1[...], preferred_element_type=jnp.float32)
               + b1[...])
    h = _leaky(jnp.dot(h, w2[...], preferred_element_type=jnp.float32)
               + b2[...])
    xo[...] = h
    hs = invs[...] * h
    hs0[...] = hs[:, :_HH]
    hs1[...] = hs[:, _HH:]


def _t1(x, w1, b1, w2, b2, invs):
    f32 = jnp.float32
    return pl.pallas_call(
        _t1_body,
        grid=(_GRID,),
        in_specs=[_row_spec(_DIN), _full_spec((_DIN, _H)), _full_spec((1, _H)),
                  _full_spec((_H, _H)), _full_spec((1, _H)), _row_spec(1)],
        out_specs=[_row_spec(_H), _row_spec(_HH), _row_spec(_HH)],
        out_shape=[jax.ShapeDtypeStruct((_N, _H), f32),
                   jax.ShapeDtypeStruct((_N, _HH), f32),
                   jax.ShapeDtypeStruct((_N, _HH), f32)],
    )(x, w1, b1, w2, b2, invs)


def _t2_body(x, v00, b00, v10, b10, v01, b01, v11, b11,
             o00, o10, o01, o11):
    xv = x[...]
    o00[...] = jnp.dot(xv, v00[...], preferred_element_type=jnp.float32) + b00[...]
    o10[...] = jnp.dot(xv, v10[...], preferred_element_type=jnp.float32) + b10[...]
    o01[...] = jnp.dot(xv, v01[...], preferred_element_type=jnp.float32) + b01[...]
    o11[...] = jnp.dot(xv, v11[...], preferred_element_type=jnp.float32) + b11[...]


def _t2(x, v00, b00, v10, b10, v01, b01, v11, b11):
    f32 = jnp.float32
    return pl.pallas_call(
        _t2_body,
        grid=(_GRID,),
        in_specs=[_row_spec(_H)] + [_full_spec((_H, _H)), _full_spec((1, _H))] * 4,
        out_specs=[_row_spec(_H)] * 4,
        out_shape=[jax.ShapeDtypeStruct((_N, _H), f32)] * 4,
    )(x, v00, b00, v10, b10, v01, b01, v11, b11)


def _t3_body(a0, a1, w, xv, invd, invs, h_o, hs0_o, hs1_o):
    wm = w[...]
    t = (jnp.dot(a0[...], wm[:_HH, :], preferred_element_type=jnp.float32)
         + jnp.dot(a1[...], wm[_HH:, :], preferred_element_type=jnp.float32))
    h = _leaky(invd[...] * t + xv[...])
    h_o[...] = h
    hs = invs[...] * h
    hs0_o[...] = hs[:, :_HH]
    hs1_o[...] = hs[:, _HH:]


def _t3(a0, a1, w, xv, invd, invs):
    f32 = jnp.float32
    return pl.pallas_call(
        _t3_body,
        grid=(_GRID,),
        in_specs=[_row_spec(_HH), _row_spec(_HH), _full_spec((_H, _H)),
                  _row_spec(_H), _row_spec(1), _row_spec(1)],
        out_specs=[_row_spec(_H), _row_spec(_HH), _row_spec(_HH)],
        out_shape=[jax.ShapeDtypeStruct((_N, _H), f32),
                   jax.ShapeDtypeStruct((_N, _HH), f32),
                   jax.ShapeDtypeStruct((_N, _HH), f32)],
    )(a0, a1, w, xv, invd, invs)


def _t4_body(h0, h1, invs, xo, hs0, hs1):
    x = (h0[...] + h1[...]) * 0.5
    xo[...] = x
    hs = invs[...] * x
    hs0[...] = hs[:, :_HH]
    hs1[...] = hs[:, _HH:]


def _t4(h0, h1, invs):
    f32 = jnp.float32
    return pl.pallas_call(
        _t4_body,
        grid=(_GRID,),
        in_specs=[_row_spec(_H), _row_spec(_H), _row_spec(1)],
        out_specs=[_row_spec(_H), _row_spec(_HH), _row_spec(_HH)],
        out_shape=[jax.ShapeDtypeStruct((_N, _H), f32),
                   jax.ShapeDtypeStruct((_N, _HH), f32),
                   jax.ShapeDtypeStruct((_N, _HH), f32)],
    )(h0, h1, invs)


def _t5_body(h0, h1, w1, b1, w2, b2, wr, br, out):
    x = (h0[...] + h1[...]) * 0.5
    x = _leaky(jnp.dot(x, w1[...], preferred_element_type=jnp.float32) + b1[...])
    x = _leaky(jnp.dot(x, w2[...], preferred_element_type=jnp.float32) + b2[...])
    out[...] = jnp.dot(x, wr[...], preferred_element_type=jnp.float32) + br[...]


def _t5(h0, h1, w1, b1, w2, b2, wr, br):
    return pl.pallas_call(
        _t5_body,
        grid=(_GRID,),
        in_specs=[_row_spec(_H), _row_spec(_H),
                  _full_spec((_H, _H)), _full_spec((1, _H)),
                  _full_spec((_H, _H)), _full_spec((1, _H)),
                  _full_spec((_H, _OUT)), _full_spec((1, _OUT))],
        out_specs=[_row_spec(_OUT)],
        out_shape=[jax.ShapeDtypeStruct((_N, _OUT), jnp.float32)],
    )(h0, h1, w1, b1, w2, b2, wr, br)[0]


def kernel(X, edge_index, pre1_W, pre1_b, pre2_W, pre2_b,
           arma_W0, arma_V0, arma_b0, arma_W1, arma_V1, arma_b1,
           post1_W, post1_b, post2_W, post2_b, ro_W, ro_b):
    f32 = jnp.float32
    src = edge_index[0]
    dst = edge_index[1]
    deg_s = jnp.zeros((_N,), f32).at[src].add(1.0)
    deg_d = jnp.zeros((_N,), f32).at[dst].add(1.0)
    inv_s = jnp.where(deg_s > 0, lax.rsqrt(jnp.maximum(deg_s, 1.0)), 0.0)
    inv_d = jnp.where(deg_d > 0, lax.rsqrt(jnp.maximum(deg_d, 1.0)), 0.0)
    invs2 = inv_s[:, None]
    invd2 = inv_d[:, None]

    src2d = src.reshape(_NCHUNK, _CH)
    dst2d = dst.reshape(_NCHUNK, _CH)

    def b2d(b):
        return b.reshape(1, -1)

    x_in, hs0, hs1 = _t1(X, pre1_W, b2d(pre1_b), pre2_W, b2d(pre2_b), invs2)

    for l in range(_LAYERS):
        xv00, xv10, xv01, xv11 = _t2(
            x_in,
            arma_V0[l, 0], b2d(arma_b0[l, 0]), arma_V1[l, 0], b2d(arma_b1[l, 0]),
            arma_V0[l, 1], b2d(arma_b0[l, 1]), arma_V1[l, 1], b2d(arma_b1[l, 1]))
        xv0 = (xv00, xv10)
        xv1 = (xv01, xv11)
        hs = [(hs0, hs1), (hs0, hs1)]
        h = [None, None]
        for t in range(_ITERS):
            for k in range(_ORDER):
                a0, a1 = _sc_prop(hs[k][0], hs[k][1], src2d, dst2d)
                W = (arma_W0 if t == 0 else arma_W1)[l, k]
                xv = (xv0 if k == 0 else xv1)[0 if t == 0 else 1]
                h[k], h0_, h1_ = _t3(a0, a1, W, xv, invd2, invs2)
                hs[k] = (h0_, h1_)
        if l < _LAYERS - 1:
            x_in, hs0, hs1 = _t4(h[0], h[1], invs2)

    return _t5(h[0], h[1], post1_W, b2d(post1_b), post2_W, b2d(post2_b),
               ro_W, b2d(ro_b))


# speedup vs baseline: 1.7349x; 1.1067x over previous
import functools

import jax
import jax.numpy as jnp
from jax import lax
from jax.experimental import pallas as pl
from jax.experimental.pallas import tpu as pltpu
from jax.experimental.pallas import tpu_sc as plsc

_N = 50000
_E = 800000
_DIN = 128
_H = 64
_HH = 32
_OUT = 8
_LAYERS = 5
_ORDER = 2
_ITERS = 8

_NP = 50176
_RPS = 3136
_ZR = 56
_CH = 125
_KG = 2
_NPAR = 3
_NCHUNK = _E // _CH
_CPS = _NCHUNK // 16
_GPS = _CPS // _KG
_GI = 8
_GO = _GPS // _GI

_BLK = 2000
_GRID = _N // _BLK


def _leaky(x):
    return jnp.where(x >= 0, x, 0.2 * x)


def _sc_prop(hs0, hs1, src2d, dst2d):
    mesh = plsc.VectorSubcoreMesh(core_axis_name="c", subcore_axis_name="s")

    @functools.partial(
        pl.kernel,
        mesh=mesh,
        compiler_params=pltpu.CompilerParams(use_tc_tiling_on_sc=False),
        out_type=[
            jax.ShapeDtypeStruct((_NP, _HH), jnp.float32),
            jax.ShapeDtypeStruct((_NP, _HH), jnp.float32),
        ],
        scratch_types=[
            pltpu.VMEM_SHARED((_NP, _HH), jnp.float32),
            pltpu.VMEM((_GI * _KG, _CH), jnp.int32),
            pltpu.VMEM((_GI * _KG, _CH), jnp.int32),
            pltpu.VMEM((_NPAR * _KG, _CH, _HH), jnp.float32),
            pltpu.VMEM((_ZR, _HH), jnp.float32),
            pltpu.SemaphoreType.DMA,
            pltpu.SemaphoreType.DMA,
            pltpu.SemaphoreType.DMA,
            pltpu.SemaphoreType.DMA,
            pltpu.SemaphoreType.DMA,
            pltpu.SemaphoreType.DMA,
            pltpu.SemaphoreType.DMA,
        ],
    )
    def kern(hs0_hbm, hs1_hbm, src_hbm, dst_hbm, o0_hbm, o1_hbm,
             acc, sidx, didx, rows, zbuf, sem_i,
             sem_g0, sem_g1, sem_g2, sem_s0, sem_s1, sem_s2):
        cid = lax.axis_index("c")
        sid = lax.axis_index("s")
        sem_g = (sem_g0, sem_g1, sem_g2)
        sem_s = (sem_s0, sem_s1, sem_s2)

        @pl.loop(0, _ZR)
        def _(r):
            zbuf[r, pl.ds(0, 16)] = jnp.zeros((16,), jnp.float32)
            zbuf[r, pl.ds(16, 16)] = jnp.zeros((16,), jnp.float32)

        def run(hs_hbm, o_hbm):
            base = sid * _RPS
            zcps = [
                pltpu.async_copy(zbuf, acc.at[pl.ds(base + i * _ZR, _ZR)],
                                 sem_s0)
                for i in range(_RPS // _ZR)
            ]
            for c in zcps:
                c.wait()

            plsc.subcore_barrier()

            c00 = sid * _CPS

            def fire_gath(g):
                p = g % _NPAR
                return [
                    pltpu.async_copy(
                        hs_hbm.at[sidx.at[g * _KG + j]],
                        rows.at[p * _KG + j], sem_g[p])
                    for j in range(_KG)
                ]

            def fire_scat(g):
                p = g % _NPAR
                return [
                    pltpu.async_copy(
                        rows.at[p * _KG + j],
                        acc.at[didx.at[g * _KG + j]],
                        sem_s[p], add=True)
                    for j in range(_KG)
                ]

            @pl.loop(0, _GO)
            def _(gi):
                gbase = c00 + gi * (_GI * _KG)
                ci_s = pltpu.async_copy(
                    src_hbm.at[pl.ds(gbase, _GI * _KG)], sidx, sem_i)
                ci_d = pltpu.async_copy(
                    dst_hbm.at[pl.ds(gbase, _GI * _KG)], didx, sem_i)
                ci_s.wait()
                ci_d.wait()
                gath = {0: fire_gath(0)}
                scat = {}
                waited = -1
                for g in range(_GI):
                    if g + 1 < _GI:
                        if g - 2 >= 0:
                            for c in scat[g - 2]:
                                c.wait()
                            waited = g - 2
                        gath[g + 1] = fire_gath(g + 1)
                    for c in gath[g]:
                        c.wait()
                    scat[g] = fire_scat(g)
                for g in range(waited + 1, _GI):
                    for c in scat[g]:
                        c.wait()

            plsc.subcore_barrier()
            pltpu.sync_copy(acc.at[pl.ds(sid * _RPS, _RPS)],
                            o_hbm.at[pl.ds(sid * _RPS, _RPS)])

        @pl.when(cid == 0)
        def _():
            run(hs0_hbm, o0_hbm)

        @pl.when(cid == 1)
        def _():
            run(hs1_hbm, o1_hbm)

    return kern(hs0, hs1, src2d, dst2d)


def _row_spec(cols):
    return pl.BlockSpec((_BLK, cols), lambda i: (i, 0))


def _full_spec(shape):
    return pl.BlockSpec(shape, lambda i: tuple(0 for _ in shape))


def _t1_body(x, w1, b1, w2, b2, invs, xo, hs0, hs1):
    h = _leaky(jnp.dot(x[...], w1[...], preferred_element_type=jnp.float32)
               + b1[...])
    h = _leaky(jnp.dot(h, w2[...], preferred_element_type=jnp.float32)
               + b2[...])
    xo[...] = h
    hs = invs[...] * h
    hs0[...] = hs[:, :_HH]
    hs1[...] = hs[:, _HH:]


def _t1(x, w1, b1, w2, b2, invs):
    f32 = jnp.float32
    return pl.pallas_call(
        _t1_body,
        grid=(_GRID,),
        in_specs=[_row_spec(_DIN), _full_spec((_DIN, _H)), _full_spec((1, _H)),
                  _full_spec((_H, _H)), _full_spec((1, _H)), _row_spec(1)],
        out_specs=[_row_spec(_H), _row_spec(_HH), _row_spec(_HH)],
        out_shape=[jax.ShapeDtypeStruct((_N, _H), f32),
                   jax.ShapeDtypeStruct((_N, _HH), f32),
                   jax.ShapeDtypeStruct((_N, _HH), f32)],
    )(x, w1, b1, w2, b2, invs)


def _t2_body(x, v00, b00, v10, b10, v01, b01, v11, b11,
             o00, o10, o01, o11):
    xv = x[...]
    o00[...] = jnp.dot(xv, v00[...], preferred_element_type=jnp.float32) + b00[...]
    o10[...] = jnp.dot(xv, v10[...], preferred_element_type=jnp.float32) + b10[...]
    o01[...] = jnp.dot(xv, v01[...], preferred_element_type=jnp.float32) + b01[...]
    o11[...] = jnp.dot(xv, v11[...], preferred_element_type=jnp.float32) + b11[...]


def _t2(x, v00, b00, v10, b10, v01, b01, v11, b11):
    f32 = jnp.float32
    return pl.pallas_call(
        _t2_body,
        grid=(_GRID,),
        in_specs=[_row_spec(_H)] + [_full_spec((_H, _H)), _full_spec((1, _H))] * 4,
        out_specs=[_row_spec(_H)] * 4,
        out_shape=[jax.ShapeDtypeStruct((_N, _H), f32)] * 4,
    )(x, v00, b00, v10, b10, v01, b01, v11, b11)


def _t3_body(a0, a1, w, xv, invd, invs, h_o, hs0_o, hs1_o):
    wm = w[...]
    t = (jnp.dot(a0[...], wm[:_HH, :], preferred_element_type=jnp.float32)
         + jnp.dot(a1[...], wm[_HH:, :], preferred_element_type=jnp.float32))
    h = _leaky(invd[...] * t + xv[...])
    h_o[...] = h
    hs = invs[...] * h
    hs0_o[...] = hs[:, :_HH]
    hs1_o[...] = hs[:, _HH:]


def _t3(a0, a1, w, xv, invd, invs):
    f32 = jnp.float32
    return pl.pallas_call(
        _t3_body,
        grid=(_GRID,),
        in_specs=[_row_spec(_HH), _row_spec(_HH), _full_spec((_H, _H)),
                  _row_spec(_H), _row_spec(1), _row_spec(1)],
        out_specs=[_row_spec(_H), _row_spec(_HH), _row_spec(_HH)],
        out_shape=[jax.ShapeDtypeStruct((_N, _H), f32),
                   jax.ShapeDtypeStruct((_N, _HH), f32),
                   jax.ShapeDtypeStruct((_N, _HH), f32)],
    )(a0, a1, w, xv, invd, invs)


def _t4_body(h0, h1, invs, xo, hs0, hs1):
    x = (h0[...] + h1[...]) * 0.5
    xo[...] = x
    hs = invs[...] * x
    hs0[...] = hs[:, :_HH]
    hs1[...] = hs[:, _HH:]


def _t4(h0, h1, invs):
    f32 = jnp.float32
    return pl.pallas_call(
        _t4_body,
        grid=(_GRID,),
        in_specs=[_row_spec(_H), _row_spec(_H), _row_spec(1)],
        out_specs=[_row_spec(_H), _row_spec(_HH), _row_spec(_HH)],
        out_shape=[jax.ShapeDtypeStruct((_N, _H), f32),
                   jax.ShapeDtypeStruct((_N, _HH), f32),
                   jax.ShapeDtypeStruct((_N, _HH), f32)],
    )(h0, h1, invs)


def _t5_body(h0, h1, w1, b1, w2, b2, wr, br, out):
    x = (h0[...] + h1[...]) * 0.5
    x = _leaky(jnp.dot(x, w1[...], preferred_element_type=jnp.float32) + b1[...])
    x = _leaky(jnp.dot(x, w2[...], preferred_element_type=jnp.float32) + b2[...])
    out[...] = jnp.dot(x, wr[...], preferred_element_type=jnp.float32) + br[...]


def _t5(h0, h1, w1, b1, w2, b2, wr, br):
    return pl.pallas_call(
        _t5_body,
        grid=(_GRID,),
        in_specs=[_row_spec(_H), _row_spec(_H),
                  _full_spec((_H, _H)), _full_spec((1, _H)),
                  _full_spec((_H, _H)), _full_spec((1, _H)),
                  _full_spec((_H, _OUT)), _full_spec((1, _OUT))],
        out_specs=[_row_spec(_OUT)],
        out_shape=[jax.ShapeDtypeStruct((_N, _OUT), jnp.float32)],
    )(h0, h1, w1, b1, w2, b2, wr, br)[0]


def kernel(X, edge_index, pre1_W, pre1_b, pre2_W, pre2_b,
           arma_W0, arma_V0, arma_b0, arma_W1, arma_V1, arma_b1,
           post1_W, post1_b, post2_W, post2_b, ro_W, ro_b):
    f32 = jnp.float32
    src = edge_index[0]
    dst = edge_index[1]
    deg_s = jnp.zeros((_N,), f32).at[src].add(1.0)
    deg_d = jnp.zeros((_N,), f32).at[dst].add(1.0)
    inv_s = jnp.where(deg_s > 0, lax.rsqrt(jnp.maximum(deg_s, 1.0)), 0.0)
    inv_d = jnp.where(deg_d > 0, lax.rsqrt(jnp.maximum(deg_d, 1.0)), 0.0)
    invs2 = inv_s[:, None]
    invd2 = inv_d[:, None]

    src2d = src.reshape(_NCHUNK, _CH)
    dst2d = dst.reshape(_NCHUNK, _CH)

    def b2d(b):
        return b.reshape(1, -1)

    x_in, hs0, hs1 = _t1(X, pre1_W, b2d(pre1_b), pre2_W, b2d(pre2_b), invs2)

    for l in range(_LAYERS):
        xv00, xv10, xv01, xv11 = _t2(
            x_in,
            arma_V0[l, 0], b2d(arma_b0[l, 0]), arma_V1[l, 0], b2d(arma_b1[l, 0]),
            arma_V0[l, 1], b2d(arma_b0[l, 1]), arma_V1[l, 1], b2d(arma_b1[l, 1]))
        xv0 = (xv00, xv10)
        xv1 = (xv01, xv11)
        hs = [(hs0, hs1), (hs0, hs1)]
        h = [None, None]
        for t in range(_ITERS):
            for k in range(_ORDER):
                a0, a1 = _sc_prop(hs[k][0], hs[k][1], src2d, dst2d)
                W = (arma_W0 if t == 0 else arma_W1)[l, k]
                xv = (xv0 if k == 0 else xv1)[0 if t == 0 else 1]
                h[k], h0_, h1_ = _t3(a0, a1, W, xv, invd2, invs2)
                hs[k] = (h0_, h1_)
        if l < _LAYERS - 1:
            x_in, hs0, hs1 = _t4(h[0], h[1], invs2)

    return _t5(h[0], h[1], post1_W, b2d(post1_b), post2_W, b2d(post2_b),
               ro_W, b2d(ro_b))


# speedup vs baseline: 1.7751x; 1.0232x over previous
import functools

import jax
import jax.numpy as jnp
from jax import lax
from jax.experimental import pallas as pl
from jax.experimental.pallas import tpu as pltpu
from jax.experimental.pallas import tpu_sc as plsc

_N = 50000
_E = 800000
_DIN = 128
_H = 64
_HH = 32
_OUT = 8
_LAYERS = 5
_ORDER = 2
_ITERS = 8

_NP = 50176
_RPS = 3136
_ZR = 32
_CH = 125
_KG = 2
_NPAR = 3
_NCHUNK = _E // _CH
_CPS = _NCHUNK // 16
_GPS = _CPS // _KG
_GI = 10
_GO = _GPS // _GI

_BLK = 2000
_GRID = _N // _BLK


def _leaky(x):
    return jnp.where(x >= 0, x, 0.2 * x)


def _sc_prop(hs0, hs1, e2d):
    mesh = plsc.VectorSubcoreMesh(core_axis_name="c", subcore_axis_name="s")

    @functools.partial(
        pl.kernel,
        mesh=mesh,
        compiler_params=pltpu.CompilerParams(use_tc_tiling_on_sc=False),
        out_type=[
            jax.ShapeDtypeStruct((_NP, _HH), jnp.float32),
            jax.ShapeDtypeStruct((_NP, _HH), jnp.float32),
        ],
        scratch_types=[
            pltpu.VMEM_SHARED((_NP, _HH), jnp.float32),
            pltpu.VMEM((2 * _GI * _KG, _CH), jnp.int32),
            pltpu.VMEM((_NPAR * _KG, _CH, _HH), jnp.float32),
            pltpu.VMEM((_ZR, _HH), jnp.float32),
            pltpu.SemaphoreType.DMA,
            pltpu.SemaphoreType.DMA,
            pltpu.SemaphoreType.DMA,
            pltpu.SemaphoreType.DMA,
            pltpu.SemaphoreType.DMA,
            pltpu.SemaphoreType.DMA,
            pltpu.SemaphoreType.DMA,
        ],
    )
    def kern(hs0_hbm, hs1_hbm, e_hbm, o0_hbm, o1_hbm,
             acc, eidx, rows, zbuf, sem_i,
             sem_g0, sem_g1, sem_g2, sem_s0, sem_s1, sem_s2):
        cid = lax.axis_index("c")
        sid = lax.axis_index("s")
        sem_g = (sem_g0, sem_g1, sem_g2)
        sem_s = (sem_s0, sem_s1, sem_s2)

        @pl.loop(0, _ZR)
        def _(r):
            zbuf[r, pl.ds(0, 16)] = jnp.zeros((16,), jnp.float32)
            zbuf[r, pl.ds(16, 16)] = jnp.zeros((16,), jnp.float32)

        def run(hs_hbm, o_hbm):
            base = sid * _RPS
            zcps = [
                pltpu.async_copy(zbuf, acc.at[pl.ds(base + i * _ZR, _ZR)],
                                 sem_s0)
                for i in range(_RPS // _ZR)
            ]
            for c in zcps:
                c.wait()

            plsc.subcore_barrier()

            c00 = sid * _CPS

            def fire_gath(g):
                p = g % _NPAR
                return [
                    pltpu.async_copy(
                        hs_hbm.at[eidx.at[2 * (g * _KG + j)]],
                        rows.at[p * _KG + j], sem_g[p])
                    for j in range(_KG)
                ]

            def fire_scat(g):
                p = g % _NPAR
                return [
                    pltpu.async_copy(
                        rows.at[p * _KG + j],
                        acc.at[eidx.at[2 * (g * _KG + j) + 1]],
                        sem_s[p], add=True)
                    for j in range(_KG)
                ]

            @pl.loop(0, _GO)
            def _(gi):
                gbase = c00 + gi * (_GI * _KG)
                pltpu.async_copy(
                    e_hbm.at[pl.ds(2 * gbase, 2 * _GI * _KG)], eidx,
                    sem_i).wait()
                gath = {0: fire_gath(0)}
                scat = {}
                waited = -1
                for g in range(_GI):
                    if g + 1 < _GI:
                        if g - 2 >= 0:
                            for c in scat[g - 2]:
                                c.wait()
                            waited = g - 2
                        gath[g + 1] = fire_gath(g + 1)
                    for c in gath[g]:
                        c.wait()
                    scat[g] = fire_scat(g)
                for g in range(waited + 1, _GI):
                    for c in scat[g]:
                        c.wait()

            plsc.subcore_barrier()
            pltpu.sync_copy(acc.at[pl.ds(sid * _RPS, _RPS)],
                            o_hbm.at[pl.ds(sid * _RPS, _RPS)])

        @pl.when(cid == 0)
        def _():
            run(hs0_hbm, o0_hbm)

        @pl.when(cid == 1)
        def _():
            run(hs1_hbm, o1_hbm)

    return kern(hs0, hs1, e2d)


def _row_spec(cols):
    return pl.BlockSpec((_BLK, cols), lambda i: (i, 0))


def _full_spec(shape):
    return pl.BlockSpec(shape, lambda i: tuple(0 for _ in shape))


def _t1_body(x, w1, b1, w2, b2, invs, xo, hs0, hs1):
    h = _leaky(jnp.dot(x[...], w1[...], preferred_element_type=jnp.float32)
               + b1[...])
    h = _leaky(jnp.dot(h, w2[...], preferred_element_type=jnp.float32)
               + b2[...])
    xo[...] = h
    hs = invs[...] * h
    hs0[...] = hs[:, :_HH]
    hs1[...] = hs[:, _HH:]


def _t1(x, w1, b1, w2, b2, invs):
    f32 = jnp.float32
    return pl.pallas_call(
        _t1_body,
        grid=(_GRID,),
        in_specs=[_row_spec(_DIN), _full_spec((_DIN, _H)), _full_spec((1, _H)),
                  _full_spec((_H, _H)), _full_spec((1, _H)), _row_spec(1)],
        out_specs=[_row_spec(_H), _row_spec(_HH), _row_spec(_HH)],
        out_shape=[jax.ShapeDtypeStruct((_N, _H), f32),
                   jax.ShapeDtypeStruct((_N, _HH), f32),
                   jax.ShapeDtypeStruct((_N, _HH), f32)],
    )(x, w1, b1, w2, b2, invs)


def _t2_body(x, v00, b00, v10, b10, v01, b01, v11, b11,
             o00, o10, o01, o11):
    xv = x[...]
    o00[...] = jnp.dot(xv, v00[...], preferred_element_type=jnp.float32) + b00[...]
    o10[...] = jnp.dot(xv, v10[...], preferred_element_type=jnp.float32) + b10[...]
    o01[...] = jnp.dot(xv, v01[...], preferred_element_type=jnp.float32) + b01[...]
    o11[...] = jnp.dot(xv, v11[...], preferred_element_type=jnp.float32) + b11[...]


def _t2(x, v00, b00, v10, b10, v01, b01, v11, b11):
    f32 = jnp.float32
    return pl.pallas_call(
        _t2_body,
        grid=(_GRID,),
        in_specs=[_row_spec(_H)] + [_full_spec((_H, _H)), _full_spec((1, _H))] * 4,
        out_specs=[_row_spec(_H)] * 4,
        out_shape=[jax.ShapeDtypeStruct((_N, _H), f32)] * 4,
    )(x, v00, b00, v10, b10, v01, b01, v11, b11)


def _t3_body(a0, a1, w, xv, invd, invs, h_o, hs0_o, hs1_o):
    wm = w[...]
    t = (jnp.dot(a0[...], wm[:_HH, :], preferred_element_type=jnp.float32)
         + jnp.dot(a1[...], wm[_HH:, :], preferred_element_type=jnp.float32))
    h = _leaky(invd[...] * t + xv[...])
    h_o[...] = h
    hs = invs[...] * h
    hs0_o[...] = hs[:, :_HH]
    hs1_o[...] = hs[:, _HH:]


def _t3(a0, a1, w, xv, invd, invs):
    f32 = jnp.float32
    return pl.pallas_call(
        _t3_body,
        grid=(_GRID,),
        in_specs=[_row_spec(_HH), _row_spec(_HH), _full_spec((_H, _H)),
                  _row_spec(_H), _row_spec(1), _row_spec(1)],
        out_specs=[_row_spec(_H), _row_spec(_HH), _row_spec(_HH)],
        out_shape=[jax.ShapeDtypeStruct((_N, _H), f32),
                   jax.ShapeDtypeStruct((_N, _HH), f32),
                   jax.ShapeDtypeStruct((_N, _HH), f32)],
    )(a0, a1, w, xv, invd, invs)


def _t4_body(h0, h1, invs, xo, hs0, hs1):
    x = (h0[...] + h1[...]) * 0.5
    xo[...] = x
    hs = invs[...] * x
    hs0[...] = hs[:, :_HH]
    hs1[...] = hs[:, _HH:]


def _t4(h0, h1, invs):
    f32 = jnp.float32
    return pl.pallas_call(
        _t4_body,
        grid=(_GRID,),
        in_specs=[_row_spec(_H), _row_spec(_H), _row_spec(1)],
        out_specs=[_row_spec(_H), _row_spec(_HH), _row_spec(_HH)],
        out_shape=[jax.ShapeDtypeStruct((_N, _H), f32),
                   jax.ShapeDtypeStruct((_N, _HH), f32),
                   jax.ShapeDtypeStruct((_N, _HH), f32)],
    )(h0, h1, invs)


def _t5_body(h0, h1, w1, b1, w2, b2, wr, br, out):
    x = (h0[...] + h1[...]) * 0.5
    x = _leaky(jnp.dot(x, w1[...], preferred_element_type=jnp.float32) + b1[...])
    x = _leaky(jnp.dot(x, w2[...], preferred_element_type=jnp.float32) + b2[...])
    out[...] = jnp.dot(x, wr[...], preferred_element_type=jnp.float32) + br[...]


def _t5(h0, h1, w1, b1, w2, b2, wr, br):
    return pl.pallas_call(
        _t5_body,
        grid=(_GRID,),
        in_specs=[_row_spec(_H), _row_spec(_H),
                  _full_spec((_H, _H)), _full_spec((1, _H)),
                  _full_spec((_H, _H)), _full_spec((1, _H)),
                  _full_spec((_H, _OUT)), _full_spec((1, _OUT))],
        out_specs=[_row_spec(_OUT)],
        out_shape=[jax.ShapeDtypeStruct((_N, _OUT), jnp.float32)],
    )(h0, h1, w1, b1, w2, b2, wr, br)[0]


def kernel(X, edge_index, pre1_W, pre1_b, pre2_W, pre2_b,
           arma_W0, arma_V0, arma_b0, arma_W1, arma_V1, arma_b1,
           post1_W, post1_b, post2_W, post2_b, ro_W, ro_b):
    f32 = jnp.float32
    src = edge_index[0]
    dst = edge_index[1]
    deg_s = jnp.zeros((_N,), f32).at[src].add(1.0)
    deg_d = jnp.zeros((_N,), f32).at[dst].add(1.0)
    inv_s = jnp.where(deg_s > 0, lax.rsqrt(jnp.maximum(deg_s, 1.0)), 0.0)
    inv_d = jnp.where(deg_d > 0, lax.rsqrt(jnp.maximum(deg_d, 1.0)), 0.0)
    invs2 = inv_s[:, None]
    invd2 = inv_d[:, None]

    e2d = jnp.stack(
        [src.reshape(_NCHUNK, _CH), dst.reshape(_NCHUNK, _CH)], axis=1,
    ).reshape(2 * _NCHUNK, _CH)

    def b2d(b):
        return b.reshape(1, -1)

    x_in, hs0, hs1 = _t1(X, pre1_W, b2d(pre1_b), pre2_W, b2d(pre2_b), invs2)

    for l in range(_LAYERS):
        xv00, xv10, xv01, xv11 = _t2(
            x_in,
            arma_V0[l, 0], b2d(arma_b0[l, 0]), arma_V1[l, 0], b2d(arma_b1[l, 0]),
            arma_V0[l, 1], b2d(arma_b0[l, 1]), arma_V1[l, 1], b2d(arma_b1[l, 1]))
        xv0 = (xv00, xv10)
        xv1 = (xv01, xv11)
        hs = [(hs0, hs1), (hs0, hs1)]
        h = [None, None]
        for t in range(_ITERS):
            for k in range(_ORDER):
                a0, a1 = _sc_prop(hs[k][0], hs[k][1], e2d)
                W = (arma_W0 if t == 0 else arma_W1)[l, k]
                xv = (xv0 if k == 0 else xv1)[0 if t == 0 else 1]
                h[k], h0_, h1_ = _t3(a0, a1, W, xv, invd2, invs2)
                hs[k] = (h0_, h1_)
        if l < _LAYERS - 1:
            x_in, hs0, hs1 = _t4(h[0], h[1], invs2)

    return _t5(h[0], h[1], post1_W, b2d(post1_b), post2_W, b2d(post2_b),
               ro_W, b2d(ro_b))
